# baseline (speedup 1.0000x reference)
import jax
import jax.numpy as jnp
from jax.experimental import pallas as pl
from jax.experimental.pallas import tpu as pltpu

_BS, _C, _N, _F = 128, 32, 512, 16
_MASK_RATIO = 0.4
_LEN_KEEP = int(_N * (1 - _MASK_RATIO))
_ROWS = _BS * _C
_MROWS = 512


def _mask_body(noise_ref, mask_ref):
    n = noise_ref.shape[1]
    r = noise_ref.shape[0]
    k = _LEN_KEEP
    noise_t = jnp.transpose(noise_ref[...])
    bits = jax.lax.bitcast_convert_type(noise_t, jnp.int32)

    ones_row = jnp.ones((1, n), jnp.float32)
    kf = jnp.float32(k)
    lo = jnp.zeros((1, r), jnp.int32)
    hi = jnp.full((1, r), jnp.int32(0x3F800000))

    def count(mask_nr):
        b = jnp.where(mask_nr, jnp.float32(1), jnp.float32(0))
        return jax.lax.dot_general(
            ones_row, b, (((1,), (0,)), ((), ())),
            preferred_element_type=jnp.float32)

    for _ in range(30):
        mid = lo + (hi - lo) // 2
        ge = count(bits <= mid) >= kf
        lo, hi = jnp.where(ge, lo, mid + 1), jnp.where(ge, mid, hi)
    v = lo

    need = kf - count(bits < v)
    idx = jax.lax.broadcasted_iota(jnp.int32, (n, r), 0)
    eq = bits == v
    lo2 = jnp.zeros((1, r), jnp.int32)
    hi2 = jnp.full((1, r), jnp.int32(n - 1))

    for _ in range(9):
        mid = lo2 + (hi2 - lo2) // 2
        ge = count(eq & (idx <= mid)) >= need
        lo2, hi2 = jnp.where(ge, lo2, mid + 1), jnp.where(ge, mid, hi2)
    t = lo2

    keep_t = (bits < v) | (eq & (idx <= t))
    masked_t = jnp.where(keep_t, jnp.float32(0.0), jnp.float32(1.0))
    mask_ref[...] = jnp.transpose(masked_t) > jnp.float32(0.5)


@jax.jit
def kernel(patch_input, noise):
    bs, c, n, f = patch_input.shape
    rows = bs * c
    noise2 = noise.reshape(rows, n)
    mask2 = pl.pallas_call(
        _mask_body,
        grid=(rows // _MROWS,),
        in_specs=[pl.BlockSpec((_MROWS, n), lambda i: (i, 0))],
        out_specs=pl.BlockSpec((_MROWS, n), lambda i: (i, 0)),
        out_shape=jax.ShapeDtypeStruct((rows, n), jnp.bool_),
    )(noise2)
    mask = mask2.reshape(bs, c, n)
    out = jnp.where(mask[..., None], jnp.float32(0.0), patch_input)
    return out, mask


# speedup vs baseline: 1.1997x; 1.1997x over previous
import jax
import jax.numpy as jnp
from jax.experimental import pallas as pl
from jax.experimental.pallas import tpu as pltpu

_BS, _C, _N, _F = 128, 32, 512, 16
_MASK_RATIO = 0.4
_LEN_KEEP = int(_N * (1 - _MASK_RATIO))
_ROWS = _BS * _C
_MROWS = 512


def _mask_body(noise_ref, mask_ref):
    n = noise_ref.shape[1]
    r = noise_ref.shape[0]
    k = _LEN_KEEP
    noise_t = jnp.transpose(noise_ref[...])
    bits = jax.lax.bitcast_convert_type(noise_t, jnp.int32)

    lo = jnp.zeros((1, r), jnp.int32)
    hi = jnp.full((1, r), jnp.int32(0x3F800000))

    def count(mask_nr):
        return jnp.sum(mask_nr.astype(jnp.int32), axis=0, keepdims=True)

    for _ in range(30):
        mid = lo + (hi - lo) // 2
        ge = count(bits <= mid) >= k
        lo, hi = jnp.where(ge, lo, mid + 1), jnp.where(ge, mid, hi)
    v = lo

    need = k - count(bits < v)
    idx = jax.lax.broadcasted_iota(jnp.int32, (n, r), 0)
    eq = bits == v
    lo2 = jnp.zeros((1, r), jnp.int32)
    hi2 = jnp.full((1, r), jnp.int32(n - 1))

    for _ in range(9):
        mid = lo2 + (hi2 - lo2) // 2
        ge = count(eq & (idx <= mid)) >= need
        lo2, hi2 = jnp.where(ge, lo2, mid + 1), jnp.where(ge, mid, hi2)
    t = lo2

    keep_t = (bits < v) | (eq & (idx <= t))
    masked_t = jnp.where(keep_t, jnp.float32(0.0), jnp.float32(1.0))
    mask_ref[...] = jnp.transpose(masked_t) > jnp.float32(0.5)


@jax.jit
def kernel(patch_input, noise):
    bs, c, n, f = patch_input.shape
    rows = bs * c
    noise2 = noise.reshape(rows, n)
    mask2 = pl.pallas_call(
        _mask_body,
        grid=(rows // _MROWS,),
        in_specs=[pl.BlockSpec((_MROWS, n), lambda i: (i, 0))],
        out_specs=pl.BlockSpec((_MROWS, n), lambda i: (i, 0)),
        out_shape=jax.ShapeDtypeStruct((rows, n), jnp.bool_),
    )(noise2)
    mask = mask2.reshape(bs, c, n)
    out = jnp.where(mask[..., None], jnp.float32(0.0), patch_input)
    return out, mask
